# baseline (speedup 1.0000x reference)
import jax
import jax.numpy as jnp
from jax import lax
from jax.experimental import pallas as pl
from jax.experimental.pallas import tpu as pltpu, tpu_sc as plsc

D = 64
NC, NS = 2, 16
NW = NC * NS
CH = 128
SCALE = 8.0


def _body(idx_hbm, table_hbm, out_hbm, idx_v, buf, gsem):
    wid = lax.axis_index("s") * NC + lax.axis_index("c")
    nch = idx_hbm.shape[1]
    per_w = nch * CH
    pltpu.sync_copy(idx_hbm.at[wid], idx_v)

    def chunk(j, carry):
        pltpu.async_copy(table_hbm.at[idx_v.at[j]], buf, gsem).wait()

        def scale(r, c2):
            for t in range(D // 16):
                sl = pl.ds(t * 16, 16)
                buf[r, sl] = buf[r, sl] * SCALE
            return c2

        lax.fori_loop(0, CH, scale, 0)
        pltpu.sync_copy(buf, out_hbm.at[pl.ds(wid * per_w + j * CH, CH)])
        return carry

    lax.fori_loop(0, nch, chunk, 0)


@jax.jit
def _embed(xf, table):
    b = xf.shape[0]
    nch = b // (NW * CH)
    idx3 = xf.reshape(NW, nch, CH)
    mesh = plsc.VectorSubcoreMesh(core_axis_name="c", subcore_axis_name="s")
    return pl.kernel(
        _body,
        out_type=jax.ShapeDtypeStruct((b, D), jnp.float32),
        mesh=mesh,
        scratch_types=[
            pltpu.VMEM((nch, CH), jnp.int32),
            pltpu.VMEM((CH, D), jnp.float32),
            pltpu.SemaphoreType.DMA,
        ],
        compiler_params=pltpu.CompilerParams(use_tc_tiling_on_sc=False),
    )(idx3, table)


def kernel(x, table):
    s, t = x.shape
    out = _embed(x.reshape(s * t), table)
    return out.reshape(s, t, D)


# speedup vs baseline: 1.1962x; 1.1962x over previous
import jax
import jax.numpy as jnp
from jax import lax
from jax.experimental import pallas as pl
from jax.experimental.pallas import tpu as pltpu, tpu_sc as plsc

D = 64
NC, NS = 2, 16
NW = NC * NS
CH = 128
NBUF = 5
PRE = 3
SCALE = 8.0


def _body(idx_hbm, table_hbm, out_hbm, idx_v, *scratch):
    bufs = scratch[:NBUF]
    gsems = scratch[NBUF:2 * NBUF]
    osems = scratch[2 * NBUF:3 * NBUF]
    wid = lax.axis_index("s") * NC + lax.axis_index("c")
    nch = idx_hbm.shape[1]
    base = wid * nch * CH

    def gather(c, b, wait=False):
        d = pltpu.make_async_copy(table_hbm.at[idx_v.at[c]], bufs[b], gsems[b])
        d.wait() if wait else d.start()

    def store(c, b, wait=False):
        d = pltpu.make_async_copy(
            bufs[b], out_hbm.at[pl.ds(base + c * CH, CH)], osems[b])
        d.wait() if wait else d.start()

    pltpu.sync_copy(idx_hbm.at[wid], idx_v)
    for c in range(PRE):
        gather(c, c % NBUF)

    def step(o, carry):
        for b in range(NBUF):
            c = o * NBUF + b
            gather(c, b, wait=True)

            @plsc.parallel_loop(0, CH, step=1, unroll=4)
            def scale(r):
                for t in range(D // 16):
                    sl = pl.ds(t * 16, 16)
                    bufs[b][r, sl] = bufs[b][r, sl] * SCALE

            store(c, b)
            f = c + PRE
            fb = (b + PRE) % NBUF

            @pl.when(f < nch)
            def _():
                @pl.when(f >= NBUF)
                def _():
                    store(f - NBUF, fb, wait=True)
                gather(f, fb)
        return carry

    lax.fori_loop(0, nch // NBUF, step, 0)
    for k in range(NBUF):
        c = nch - NBUF + k
        store(c, c % NBUF, wait=True)


@jax.jit
def _embed(xf, table):
    b = xf.shape[0]
    nch = b // (NW * CH)
    idx3 = xf.reshape(NW, nch, CH)
    mesh = plsc.VectorSubcoreMesh(core_axis_name="c", subcore_axis_name="s")
    return pl.kernel(
        _body,
        out_type=jax.ShapeDtypeStruct((b, D), jnp.float32),
        mesh=mesh,
        scratch_types=(
            [pltpu.VMEM((nch, CH), jnp.int32)]
            + [pltpu.VMEM((CH, D), jnp.float32) for _ in range(NBUF)]
            + [pltpu.SemaphoreType.DMA for _ in range(2 * NBUF)]
        ),
        compiler_params=pltpu.CompilerParams(use_tc_tiling_on_sc=False),
    )(idx3, table)


def kernel(x, table):
    s, t = x.shape
    out = _embed(x.reshape(s * t), table)
    return out.reshape(s, t, D)
